# baseline (speedup 1.0000x reference)
import functools

import jax
import jax.numpy as jnp
from jax import lax
from jax.experimental import pallas as pl
from jax.experimental.pallas import tpu as pltpu
from jax.experimental.pallas import tpu_sc as plsc

NC = 2
NS = 16
NW = NC * NS
NBUF = 4
NSP = 3


def _gather_body(b_per_w, ch, seq, d_model, table_hbm, idx_hbm, out_hbm,
                 idx_v, spmem, *bufs_and_sems):
    bufs = bufs_and_sems[:NBUF]
    gsems = bufs_and_sems[NBUF:2 * NBUF]
    xsems = bufs_and_sems[2 * NBUF:3 * NBUF]
    osems = bufs_and_sems[3 * NBUF:3 * NBUF + NSP]
    sid = lax.axis_index("s")
    wid = sid * NC + lax.axis_index("c")
    base = wid * b_per_w
    w_per_row = seq // b_per_w
    pltpu.sync_copy(
        idx_hbm.at[wid // w_per_row,
                   pl.ds((wid % w_per_row) * b_per_w, b_per_w)], idx_v)
    n_chunks = b_per_w // ch

    def gather(t, b):
        return pltpu.make_async_copy(
            table_hbm.at[idx_v.at[pl.ds(t * ch, ch)]], bufs[b], gsems[b])

    def stage(t, b):
        return pltpu.make_async_copy(bufs[b], spmem.at[sid, t % NSP],
                                     xsems[b])

    def out(t, b):
        return pltpu.make_async_copy(
            spmem.at[sid, t % NSP], out_hbm.at[pl.ds(base + t * ch, ch)],
            osems[t % NSP])

    gather(0, 0).start()
    gather(1, 1).start()
    gather(2, 2).start()
    for t in range(n_chunks):
        b = t % NBUF
        if t - NSP >= 0:
            out(t - NSP, (t - NSP) % NBUF).wait()
        gather(t, b).wait()
        stage(t, b).start()
        if t - 1 >= 0:
            pb = (t - 1) % NBUF
            stage(t - 1, pb).wait()
            out(t - 1, pb).start()
        if t + 3 < n_chunks:
            gather(t + 3, (t + 3) % NBUF).start()
    lb = (n_chunks - 1) % NBUF
    stage(n_chunks - 1, lb).wait()
    out(n_chunks - 1, lb).start()
    for t in range(n_chunks - NSP, n_chunks):
        out(t, t % NBUF).wait()


@functools.cache
def _make_gather(v, d_model, batch, seq):
    b_total = batch * seq
    assert b_total % (8 * NW) == 0
    b_per_w = b_total // NW
    assert seq % b_per_w == 0
    ch = 8
    assert b_per_w % ch == 0 and ch <= 128
    mesh = plsc.VectorSubcoreMesh(core_axis_name="c", subcore_axis_name="s")
    return pl.kernel(
        functools.partial(_gather_body, b_per_w, ch, seq, d_model),
        out_type=jax.ShapeDtypeStruct((b_total, d_model), jnp.float32),
        mesh=mesh,
        scratch_types=(
            [pltpu.VMEM((b_per_w,), jnp.int32),
             pltpu.VMEM_SHARED((NS, NSP, ch, d_model), jnp.float32)]
            + [pltpu.VMEM((ch, d_model), jnp.float32)] * NBUF
            + [pltpu.SemaphoreType.DMA] * (2 * NBUF + NSP)
        ),
    )


def kernel(x, table):
    batch, seq = x.shape
    v, d_model = table.shape
    out = _make_gather(v, d_model, batch, seq)(table, x.astype(jnp.int32))
    return out.reshape(batch, seq, d_model)


# speedup vs baseline: 1.0073x; 1.0073x over previous
import functools

import jax
import jax.numpy as jnp
from jax import lax
from jax.experimental import pallas as pl
from jax.experimental.pallas import tpu as pltpu
from jax.experimental.pallas import tpu_sc as plsc

NC = 2
NS = 16
NW = NC * NS
NBUF = 4
NSP = 3


def _gather_body(b_per_w, ch, seq, d_model, table_hbm, idx_hbm, out_hbm,
                 idx_v, spmem, *bufs_and_sems):
    bufs = bufs_and_sems[:NBUF]
    gsems = bufs_and_sems[NBUF:2 * NBUF]
    xsems = bufs_and_sems[2 * NBUF:3 * NBUF]
    dsems = bufs_and_sems[3 * NBUF:4 * NBUF]
    osems = bufs_and_sems[4 * NBUF:4 * NBUF + NSP]
    sid = lax.axis_index("s")
    wid = sid * NC + lax.axis_index("c")
    base = wid * b_per_w
    w_per_row = seq // b_per_w
    pltpu.sync_copy(
        idx_hbm.at[wid // w_per_row,
                   pl.ds((wid % w_per_row) * b_per_w, b_per_w)], idx_v)
    n_chunks = b_per_w // ch

    def staged(t):
        return t % 2 == 0

    def js(t):
        return t // 2

    def gather(t, b):
        return pltpu.make_async_copy(
            table_hbm.at[idx_v.at[pl.ds(t * ch, ch)]], bufs[b], gsems[b])

    def stage(t, b):
        return pltpu.make_async_copy(bufs[b], spmem.at[sid, js(t) % NSP],
                                     xsems[b])

    def out(t):
        return pltpu.make_async_copy(
            spmem.at[sid, js(t) % NSP],
            out_hbm.at[pl.ds(base + t * ch, ch)], osems[js(t) % NSP])

    def dscat(t, b):
        return pltpu.make_async_copy(
            bufs[b], out_hbm.at[pl.ds(base + t * ch, ch)], dsems[b])

    gather(0, 0).start()
    gather(1, 1).start()
    gather(2, 2).start()
    for t in range(n_chunks):
        b = t % NBUF
        if staged(t) and js(t) - NSP >= 0:
            out(2 * (js(t) - NSP)).wait()
        gather(t, b).wait()
        if staged(t):
            stage(t, b).start()
        else:
            dscat(t, b).start()
        if t >= 1:
            pb = (t - 1) % NBUF
            if staged(t - 1):
                stage(t - 1, pb).wait()
                out(t - 1).start()
            else:
                dscat(t - 1, pb).wait()
        if t + 3 < n_chunks:
            gather(t + 3, (t + 3) % NBUF).start()
    last = n_chunks - 1
    lb = last % NBUF
    if staged(last):
        stage(last, lb).wait()
        out(last).start()
    else:
        dscat(last, lb).wait()
    n_staged = (n_chunks + 1) // 2
    for j in range(n_staged - NSP, n_staged):
        out(2 * j).wait()


@functools.cache
def _make_gather(v, d_model, batch, seq):
    b_total = batch * seq
    assert b_total % (8 * NW) == 0
    b_per_w = b_total // NW
    assert seq % b_per_w == 0
    ch = 8
    assert b_per_w % (2 * ch) == 0 and ch <= 128
    mesh = plsc.VectorSubcoreMesh(core_axis_name="c", subcore_axis_name="s")
    return pl.kernel(
        functools.partial(_gather_body, b_per_w, ch, seq, d_model),
        out_type=jax.ShapeDtypeStruct((b_total, d_model), jnp.float32),
        mesh=mesh,
        scratch_types=(
            [pltpu.VMEM((b_per_w,), jnp.int32),
             pltpu.VMEM_SHARED((NS, NSP, ch, d_model), jnp.float32)]
            + [pltpu.VMEM((ch, d_model), jnp.float32)] * NBUF
            + [pltpu.SemaphoreType.DMA] * (3 * NBUF + NSP)
        ),
    )


def kernel(x, table):
    batch, seq = x.shape
    v, d_model = table.shape
    out = _make_gather(v, d_model, batch, seq)(table, x.astype(jnp.int32))
    return out.reshape(batch, seq, d_model)


# speedup vs baseline: 1.0082x; 1.0008x over previous
import functools

import jax
import jax.numpy as jnp
from jax import lax
from jax.experimental import pallas as pl
from jax.experimental.pallas import tpu as pltpu
from jax.experimental.pallas import tpu_sc as plsc

NC = 2
NS = 16
NW = NC * NS
NBUF = 4
NSP = 3
SPLIT = 3


def _gather_body(b_per_w, ch, seq, d_model, table_hbm, idx_hbm, out_hbm,
                 idx_v, spmem, *bufs_and_sems):
    bufs = bufs_and_sems[:NBUF]
    gsems = bufs_and_sems[NBUF:2 * NBUF]
    xsems = bufs_and_sems[2 * NBUF:3 * NBUF]
    dsems = bufs_and_sems[3 * NBUF:4 * NBUF]
    osems = bufs_and_sems[4 * NBUF:4 * NBUF + NSP]
    sid = lax.axis_index("s")
    wid = sid * NC + lax.axis_index("c")
    base = wid * b_per_w
    w_per_row = seq // b_per_w
    pltpu.sync_copy(
        idx_hbm.at[wid // w_per_row,
                   pl.ds((wid % w_per_row) * b_per_w, b_per_w)], idx_v)
    n_chunks = b_per_w // ch

    def staged(t):
        return t % SPLIT == 0

    def js(t):
        return t // SPLIT

    def gather(t, b):
        return pltpu.make_async_copy(
            table_hbm.at[idx_v.at[pl.ds(t * ch, ch)]], bufs[b], gsems[b])

    def stage(t, b):
        return pltpu.make_async_copy(bufs[b], spmem.at[sid, js(t) % NSP],
                                     xsems[b])

    def out(t):
        return pltpu.make_async_copy(
            spmem.at[sid, js(t) % NSP],
            out_hbm.at[pl.ds(base + t * ch, ch)], osems[js(t) % NSP])

    def dscat(t, b):
        return pltpu.make_async_copy(
            bufs[b], out_hbm.at[pl.ds(base + t * ch, ch)], dsems[b])

    gather(0, 0).start()
    gather(1, 1).start()
    gather(2, 2).start()
    for t in range(n_chunks):
        b = t % NBUF
        if staged(t) and js(t) - NSP >= 0:
            out(SPLIT * (js(t) - NSP)).wait()
        gather(t, b).wait()
        if staged(t):
            stage(t, b).start()
        else:
            dscat(t, b).start()
        if t >= 1:
            pb = (t - 1) % NBUF
            if staged(t - 1):
                stage(t - 1, pb).wait()
                out(t - 1).start()
            else:
                dscat(t - 1, pb).wait()
        if t + 3 < n_chunks:
            gather(t + 3, (t + 3) % NBUF).start()
    last = n_chunks - 1
    lb = last % NBUF
    if staged(last):
        stage(last, lb).wait()
        out(last).start()
    else:
        dscat(last, lb).wait()
    n_staged = (n_chunks + SPLIT - 1) // SPLIT
    for j in range(max(0, n_staged - NSP), n_staged):
        out(SPLIT * j).wait()


@functools.cache
def _make_gather(v, d_model, batch, seq):
    b_total = batch * seq
    assert b_total % (8 * NW) == 0
    b_per_w = b_total // NW
    assert seq % b_per_w == 0
    ch = 8
    assert b_per_w % (2 * ch) == 0 and ch <= 128
    mesh = plsc.VectorSubcoreMesh(core_axis_name="c", subcore_axis_name="s")
    return pl.kernel(
        functools.partial(_gather_body, b_per_w, ch, seq, d_model),
        out_type=jax.ShapeDtypeStruct((b_total, d_model), jnp.float32),
        mesh=mesh,
        scratch_types=(
            [pltpu.VMEM((b_per_w,), jnp.int32),
             pltpu.VMEM_SHARED((NS, NSP, ch, d_model), jnp.float32)]
            + [pltpu.VMEM((ch, d_model), jnp.float32)] * NBUF
            + [pltpu.SemaphoreType.DMA] * (3 * NBUF + NSP)
        ),
    )


def kernel(x, table):
    batch, seq = x.shape
    v, d_model = table.shape
    out = _make_gather(v, d_model, batch, seq)(table, x.astype(jnp.int32))
    return out.reshape(batch, seq, d_model)
